# Initial kernel scaffold; baseline (speedup 1.0000x reference)
import jax
import jax.numpy as jnp
from jax.experimental import pallas as pl


def kernel(edge_index_SH, edge_index_SS, edge_index_HH, prescription, SH_table, W1, b1, W2, b2, W1h, b1h, W2h, b2h, Wm, bm, gamma, beta):
    raise NotImplementedError("write your pallas kernel here")


# speedup vs baseline: 20.2112x; 20.2112x over previous
import jax
import jax.numpy as jnp
from jax import lax
from jax.experimental import pallas as pl
from jax.experimental.pallas import tpu as pltpu
from jax.experimental.pallas import tpu_sc as plsc

_N = 1195
_NU = 805
_NI = 390
_D = 64
_E = 262144
_NC = 2
_NS = 16
_NW = _NC * _NS
_EPW = _E // _NW
_CHUNK = 128
_ZC = 8192
_S = 90112
_PAD = _NS * _S


def _sc_body(edges, out, a_sh, src_v, dst_v, idx_v, ones_v, zero_v):
    c = lax.axis_index("c")
    s = lax.axis_index("s")
    wid = s * _NC + c
    base = wid * _EPW

    zvec = jnp.zeros((16,), jnp.float32)

    def _zfill(i, carry):
        zero_v[pl.ds(i * 16, 16)] = zvec
        return carry

    lax.fori_loop(0, _ZC // 16, _zfill, 0)
    ovec = jnp.ones((16,), jnp.float32)
    for i in range(_CHUNK // 16):
        ones_v[pl.ds(i * 16, 16)] = ovec

    for i in range(_S // _ZC):
        pltpu.sync_copy(zero_v, a_sh.at[pl.ds(s * _S + i * _ZC, _ZC)])

    pltpu.sync_copy(edges.at[0, pl.ds(base, _EPW)], src_v)
    pltpu.sync_copy(edges.at[1, pl.ds(base, _EPW)], dst_v)

    def _ifill(k, carry):
        j = k // (_CHUNK // 16)
        l = k % (_CHUNK // 16)
        d = dst_v[pl.ds(k * 16, 16)]
        r = src_v[pl.ds(k * 16, 16)]
        idx_v[j, pl.ds(l * 16, 16)] = d * _N + r
        return carry

    lax.fori_loop(0, _EPW // 16, _ifill, 0)

    plsc.subcore_barrier()

    for j in range(_EPW // _CHUNK):
        pltpu.sync_copy(ones_v, a_sh.at[idx_v.at[j]], add=True)

    plsc.subcore_barrier()

    pltpu.sync_copy(a_sh.at[pl.ds(s * _S, _S)], out.at[c, pl.ds(s * _S, _S)])


_sc_build_counts = pl.kernel(
    _sc_body,
    out_type=jax.ShapeDtypeStruct((_NC, _PAD), jnp.float32),
    mesh=plsc.VectorSubcoreMesh(core_axis_name="c", subcore_axis_name="s"),
    scratch_types=[
        pltpu.VMEM_SHARED((_PAD,), jnp.float32),
        pltpu.VMEM((_EPW,), jnp.int32),
        pltpu.VMEM((_EPW,), jnp.int32),
        pltpu.VMEM((_EPW // _CHUNK, _CHUNK), jnp.int32),
        pltpu.VMEM((_CHUNK,), jnp.float32),
        pltpu.VMEM((_ZC,), jnp.float32),
    ],
)


def _tc_body(a0, a1, x, w1, b1, w2, b2, cu_out, ci_out):
    f32 = jnp.float32

    def dot(u, v, prec=lax.Precision.DEFAULT):
        return lax.dot_general(u, v, (((1,), (0,)), ((), ())),
                               precision=prec, preferred_element_type=f32)

    def dot_t(u, v):
        return lax.dot_general(u, v, (((1,), (1,)), ((), ())),
                               precision=lax.Precision.DEFAULT,
                               preferred_element_type=f32)

    def agg(u, v):
        return dot(u, v, prec=lax.Precision.HIGHEST)

    a = a0[...] + a1[...]
    cnt = jnp.sum(a, axis=1, keepdims=True)
    inv = 1.0 / jnp.maximum(cnt, 1.0)

    x2 = jnp.tanh((agg(a, dot_t(x[...], w1[...])) + cnt * b1[...]) * inv)
    x6 = jnp.tanh((agg(a, dot_t(x2, w2[...])) + cnt * b2[...]) * inv)

    c0 = x6[:_NU]
    c1 = x6[_NU:]
    cu_out[...] = c1 / jnp.maximum(
        jnp.sqrt(jnp.sum(c1 * c1, axis=1, keepdims=True)), 1e-12)
    ci_out[...] = c0 / jnp.maximum(
        jnp.sqrt(jnp.sum(c0 * c0, axis=1, keepdims=True)), 1e-12)


def _tc_final_body(e, ci, out):
    out[...] = lax.dot_general(e[...], ci[...], (((1,), (1,)), ((), ())),
                               precision=lax.Precision.DEFAULT,
                               preferred_element_type=jnp.float32)


def kernel(edge_index_SH, edge_index_SS, edge_index_HH, prescription, SH_table,
           W1, b1, W2, b2, W1h, b1h, W2h, b2h, Wm, bm, gamma, beta):
    del edge_index_SS, edge_index_HH, W1h, b1h, W2h, b2h
    edges = edge_index_SH.astype(jnp.int32)
    parts = _sc_build_counts(edges)
    a0 = parts[0, :_N * _N].reshape(_N, _N)
    a1 = parts[1, :_N * _N].reshape(_N, _N)
    cu, ci = pl.pallas_call(
        _tc_body,
        out_shape=[jax.ShapeDtypeStruct((_NI, _D), jnp.float32),
                   jax.ShapeDtypeStruct((_NU, _D), jnp.float32)],
    )(a0, a1, SH_table, W1, b1.reshape(1, _D), W2, b2.reshape(1, _D))

    e_synd = prescription @ cu
    preSum = jnp.sum(prescription, axis=1, keepdims=True)
    e = e_synd / preSum
    e = e @ Wm.T + bm
    mu = jnp.mean(e, axis=0)
    var = jnp.var(e, axis=0)
    e = (e - mu) / jnp.sqrt(var + 1e-5) * gamma + beta
    e = jax.nn.relu(e)
    return pl.pallas_call(
        _tc_final_body,
        out_shape=jax.ShapeDtypeStruct((prescription.shape[0], _NU), jnp.float32),
    )(e, ci)
